# baseline (speedup 1.0000x reference)
import jax
import jax.numpy as jnp
from jax import lax
from jax.experimental import pallas as pl
from jax.experimental.pallas import tpu as pltpu
from jax.experimental.pallas import tpu_sc as plsc

NUM_FIELDS = 26
FIELD_SIZE = 100000
BATCH = 16384
NUM_CORES = 2
NUM_SUBCORES = 16
NUM_WORKERS = NUM_CORES * NUM_SUBCORES
ROWS_PER_WORKER = BATCH // NUM_WORKERS
LANES = 16
CHUNKS = ROWS_PER_WORKER // LANES
TOTAL = NUM_FIELDS * FIELD_SIZE


def _sc_body(xr_hbm, table_hbm, bias_hbm, out_hbm,
             xv, gv, accv, biasv, sems):
    wid = lax.axis_index("s") * NUM_CORES + lax.axis_index("c")
    base = wid * ROWS_PER_WORKER

    stage = []
    for f in range(NUM_FIELDS):
        fsl = pl.ds(f * ROWS_PER_WORKER, ROWS_PER_WORKER)
        stage.append(pltpu.async_copy(
            xr_hbm.at[f, wid], xv.at[fsl], sems.at[f]))
    pltpu.sync_copy(bias_hbm, biasv.at[pl.ds(0, 1)])

    table_1d = table_hbm.at[0]
    gather = []
    for f in range(NUM_FIELDS):
        fsl = pl.ds(f * ROWS_PER_WORKER, ROWS_PER_WORKER)
        stage[f].wait()
        window = table_1d.at[pl.ds(f * FIELD_SIZE, FIELD_SIZE)]
        gather.append(pltpu.async_copy(
            window.at[xv.at[fsl]], gv.at[fsl], sems.at[f]))

    bias_s = biasv[...][0]

    for f in range(NUM_FIELDS):
        gather[f].wait()
        off = f * ROWS_PER_WORKER

        if f == 0:
            def col(c, carry):
                sl = pl.ds(c * LANES, LANES)
                accv[sl] = gv[pl.ds(off + c * LANES, LANES)]
                return carry
        elif f < NUM_FIELDS - 1:
            def col(c, carry, off=off):
                sl = pl.ds(c * LANES, LANES)
                accv[sl] = accv[sl] + gv[pl.ds(off + c * LANES, LANES)]
                return carry
        else:
            def col(c, carry, off=off):
                sl = pl.ds(c * LANES, LANES)
                t = accv[sl] + gv[pl.ds(off + c * LANES, LANES)] + bias_s
                accv[sl] = 1.0 / (1.0 + jnp.exp(-t))
                return carry

        lax.fori_loop(0, CHUNKS, col, 0)

    pltpu.sync_copy(accv, out_hbm.at[pl.ds(base, ROWS_PER_WORKER)])


def kernel(x, table, bias):
    x = x.astype(jnp.int32)
    xr = x.T.reshape(NUM_FIELDS, NUM_WORKERS, ROWS_PER_WORKER)
    table_row = table.reshape(1, TOTAL)

    run = pl.kernel(
        _sc_body,
        out_type=jax.ShapeDtypeStruct((BATCH,), jnp.float32),
        mesh=plsc.VectorSubcoreMesh(core_axis_name="c", subcore_axis_name="s"),
        scratch_types=[
            pltpu.VMEM((NUM_FIELDS * ROWS_PER_WORKER,), jnp.int32),
            pltpu.VMEM((NUM_FIELDS * ROWS_PER_WORKER,), jnp.float32),
            pltpu.VMEM((ROWS_PER_WORKER,), jnp.float32),
            pltpu.VMEM((LANES,), jnp.float32),
            pltpu.SemaphoreType.DMA((NUM_FIELDS,)),
        ],
    )
    return run(xr, table_row, bias.astype(jnp.float32))


# speedup vs baseline: 1.0255x; 1.0255x over previous
import jax
import jax.numpy as jnp
from jax import lax
from jax.experimental import pallas as pl
from jax.experimental.pallas import tpu as pltpu
from jax.experimental.pallas import tpu_sc as plsc

NUM_FIELDS = 26
FIELD_SIZE = 100000
BATCH = 16384
NUM_CORES = 2
NUM_SUBCORES = 16
NUM_WORKERS = NUM_CORES * NUM_SUBCORES
ROWS_PER_WORKER = BATCH // NUM_WORKERS
LANES = 16
CHUNKS = ROWS_PER_WORKER // LANES
TOTAL = NUM_FIELDS * FIELD_SIZE


def _sc_body(xr_hbm, table_hbm, bias_hbm, out_hbm,
             xv, gv, accv, biasv, sems, stage_sems):
    wid = lax.axis_index("s") * NUM_CORES + lax.axis_index("c")
    base = wid * ROWS_PER_WORKER

    half = (NUM_FIELDS // 2) * ROWS_PER_WORKER
    stage = [
        pltpu.async_copy(xr_hbm.at[wid, pl.ds(0, half)],
                         xv.at[pl.ds(0, half)], stage_sems.at[0]),
        pltpu.async_copy(xr_hbm.at[wid, pl.ds(half, half)],
                         xv.at[pl.ds(half, half)], stage_sems.at[1]),
    ]
    pltpu.sync_copy(bias_hbm, biasv.at[pl.ds(0, 1)])

    table_1d = table_hbm.at[0]
    gather = []
    for f in range(NUM_FIELDS):
        if f in (0, NUM_FIELDS // 2):
            stage[0 if f == 0 else 1].wait()
        fsl = pl.ds(f * ROWS_PER_WORKER, ROWS_PER_WORKER)
        window = table_1d.at[pl.ds(f * FIELD_SIZE, FIELD_SIZE)]
        gather.append(pltpu.async_copy(
            window.at[xv.at[fsl]], gv.at[fsl], sems.at[f]))

    bias_s = biasv[...][0]

    for f in range(NUM_FIELDS):
        gather[f].wait()
        off = f * ROWS_PER_WORKER

        if f == 0:
            def col(c, carry):
                sl = pl.ds(c * LANES, LANES)
                accv[sl] = gv[pl.ds(off + c * LANES, LANES)]
                return carry
        elif f < NUM_FIELDS - 1:
            def col(c, carry, off=off):
                sl = pl.ds(c * LANES, LANES)
                accv[sl] = accv[sl] + gv[pl.ds(off + c * LANES, LANES)]
                return carry
        else:
            def col(c, carry, off=off):
                sl = pl.ds(c * LANES, LANES)
                t = accv[sl] + gv[pl.ds(off + c * LANES, LANES)] + bias_s
                accv[sl] = 1.0 / (1.0 + jnp.exp(-t))
                return carry

        lax.fori_loop(0, CHUNKS, col, 0)

    pltpu.sync_copy(accv, out_hbm.at[pl.ds(base, ROWS_PER_WORKER)])


def kernel(x, table, bias):
    x = x.astype(jnp.int32)
    xr = (x.T.reshape(NUM_FIELDS, NUM_WORKERS, ROWS_PER_WORKER)
           .transpose(1, 0, 2)
           .reshape(NUM_WORKERS, NUM_FIELDS * ROWS_PER_WORKER))
    table_row = table.reshape(1, TOTAL)

    run = pl.kernel(
        _sc_body,
        out_type=jax.ShapeDtypeStruct((BATCH,), jnp.float32),
        mesh=plsc.VectorSubcoreMesh(core_axis_name="c", subcore_axis_name="s"),
        scratch_types=[
            pltpu.VMEM((NUM_FIELDS * ROWS_PER_WORKER,), jnp.int32),
            pltpu.VMEM((NUM_FIELDS * ROWS_PER_WORKER,), jnp.float32),
            pltpu.VMEM((ROWS_PER_WORKER,), jnp.float32),
            pltpu.VMEM((LANES,), jnp.float32),
            pltpu.SemaphoreType.DMA((NUM_FIELDS,)),
            pltpu.SemaphoreType.DMA((2,)),
        ],
    )
    return run(xr, table_row, bias.astype(jnp.float32))
